# baseline (speedup 1.0000x reference)
import functools

import jax
import jax.numpy as jnp
from jax import lax
from jax.experimental import pallas as pl
from jax.experimental.pallas import tpu as pltpu
from jax.experimental.pallas import tpu_sc as plsc

EMB_DIM = 128
BATCH = 4096
NEG = 20
NCHUNK = NEG + 1
NW = 32
BPW = BATCH // NW
NA = 16
NRING = 4


def _sc_scores(posu2, posv2, neg3, u_table, v_table):
    mesh = plsc.VectorSubcoreMesh(core_axis_name="c", subcore_axis_name="s")

    @functools.partial(
        pl.kernel,
        mesh=mesh,
        out_type=jax.ShapeDtypeStruct((NW, NCHUNK * BPW), jnp.float32),
        compiler_params=pltpu.CompilerParams(needs_layout_passes=False),
        scratch_types=[
            pltpu.VMEM((BPW,), jnp.int32),
            pltpu.VMEM((BPW,), jnp.int32),
            pltpu.VMEM((NEG, BPW), jnp.int32),
            pltpu.VMEM((BPW, EMB_DIM), jnp.float32),
            pltpu.VMEM((BPW, EMB_DIM), jnp.float32),
            pltpu.VMEM((BPW, EMB_DIM), jnp.float32),
            pltpu.VMEM((BPW, EMB_DIM), jnp.float32),
            pltpu.VMEM((BPW, EMB_DIM), jnp.float32),
            pltpu.VMEM((BPW, EMB_DIM), jnp.float32),
            pltpu.VMEM((8, 16, 17), jnp.float32),
            pltpu.VMEM((NCHUNK * BPW,), jnp.float32),
            pltpu.SemaphoreType.DMA,
            pltpu.SemaphoreType.DMA,
            pltpu.SemaphoreType.DMA,
            pltpu.SemaphoreType.DMA,
            pltpu.SemaphoreType.DMA,
            pltpu.SemaphoreType.DMA,
        ],
    )
    def k(posu_hbm, posv_hbm, neg_hbm, u_hbm, v_hbm, out_hbm,
          idxu, idxv, negidx, urows, vrows, nbuf0, nbuf1, nbuf2, nbuf3,
          accs, scores, semu, semv, sem0, sem1, sem2, sem3):
        wid = lax.axis_index("s") * 2 + lax.axis_index("c")

        pltpu.sync_copy(posu_hbm.at[wid], idxu)
        pltpu.sync_copy(posv_hbm.at[wid], idxv)
        pltpu.sync_copy(neg_hbm.at[wid], negidx)

        ucopy = pltpu.make_async_copy(u_hbm.at[idxu], urows, semu)
        vcopy = pltpu.make_async_copy(v_hbm.at[idxv], vrows, semv)
        ucopy.start()
        vcopy.start()

        def ngather(c, buf, sem):
            return pltpu.make_async_copy(v_hbm.at[negidx.at[c]], buf, sem)

        bufs = [nbuf0, nbuf1, nbuf2, nbuf3]
        sems = [sem0, sem1, sem2, sem3]
        for s in range(NRING):
            ngather(s, bufs[s], sems[s]).start()

        with jax.named_scope("uv_wait"):
            ucopy.wait()
            vcopy.wait()

        lane = lax.iota(jnp.int32, 16)

        def col_reduce(slot):
            cols = [
                plsc.load_gather(
                    accs.at[slot], [lane, jnp.full((16,), j, jnp.int32)])
                for j in range(16)
            ]
            while len(cols) > 1:
                cols = [cols[i] + cols[i + 1] for i in range(0, len(cols), 2)]
            return cols[0]

        def tree8(prods):
            s = [prods[2 * i] + prods[2 * i + 1] for i in range(4)]
            return (s[0] + s[1]) + (s[2] + s[3])

        def dot_row(slot, l, uu, buf, r):
            accs[slot, l, pl.ds(0, 16)] = tree8(
                [uu[q] * buf[r, pl.ds(16 * q, 16)] for q in range(8)])

        with jax.named_scope("pos_loop"):
            @plsc.parallel_loop(0, BPW // 16, unroll=4)
            def pos_group(g):
                base = g * 16
                for l in range(16):
                    b = base + l
                    accs[g, l, pl.ds(0, 16)] = tree8([
                        urows[b, pl.ds(16 * q, 16)]
                        * vrows[b, pl.ds(16 * q, 16)]
                        for q in range(8)])
                scores[pl.ds(base, 16)] = col_reduce(g)

        def neg_group_a(c, buf):
            @plsc.parallel_loop(0, BPW // 16, unroll=4)
            def group(g):
                b = 8 * c + g
                uu = [urows[b, pl.ds(16 * q, 16)] for q in range(8)]
                for l in range(16):
                    dot_row(g, l, uu, buf, 16 * g + l)
                scores[pl.ds(BPW + BPW * c + 16 * g, 16)] = col_reduce(g)

        def neg_group_b(c, buf):
            @plsc.parallel_loop(0, BPW // 16, unroll=4)
            def group(g):
                b0 = 4 * (8 * (c - NA) + g)
                for i in range(4):
                    uu = [urows[b0 + i, pl.ds(16 * q, 16)] for q in range(8)]
                    for j in range(4):
                        l = 4 * i + j
                        dot_row(g, l, uu, buf, 16 * g + l)
                scores[pl.ds(BPW + BPW * c + 16 * g, 16)] = col_reduce(g)

        def body_a(i, _):
            c0 = NRING * i
            for s in range(NRING):
                c = c0 + s
                ngather(c, bufs[s], sems[s]).wait()
                neg_group_a(c, bufs[s])

                @pl.when(c + NRING < NEG)
                def _():
                    ngather(c + NRING, bufs[s], sems[s]).start()
            return 0

        with jax.named_scope("neg_a_loop"):
            lax.fori_loop(0, NA // NRING, body_a, 0)

        with jax.named_scope("neg_b"):
            for s in range(NRING):
                c = NA + s
                ngather(c, bufs[s], sems[s]).wait()
                neg_group_b(c, bufs[s])

        with jax.named_scope("writeback"):
            pltpu.sync_copy(scores, out_hbm.at[wid])

    return k(posu2, posv2, neg3, u_table, v_table)


def _finalize_kernel(s_ref, o_ref):
    x = s_ref[...]
    rows = lax.broadcasted_iota(jnp.int32, x.shape, 0)
    is_pos = (rows % NCHUNK) == 0
    xc = jnp.clip(x, -10.0, 10.0)
    p = -jax.nn.log_sigmoid(xc)
    p = -jax.nn.log_sigmoid(jnp.clip(p, -10.0, 10.0))
    n = -jax.nn.log_sigmoid(-xc)
    val = jnp.where(is_pos, p, n)
    o_ref[0, 0] = jnp.sum(val) / BATCH


def kernel(pos_u, pos_v, neg_v, u_embeddings, v_embeddings):
    pos_u = pos_u.astype(jnp.int32)
    pos_v = pos_v.astype(jnp.int32)
    neg_v = neg_v.astype(jnp.int32)

    na = neg_v[:, :16].reshape(NW, NA, BPW)
    nb = neg_v[:, 16:].reshape(NW, NEG - NA, BPW)
    neg3 = jnp.concatenate([na, nb], axis=1)

    raw = _sc_scores(
        pos_u.reshape(NW, BPW),
        pos_v.reshape(NW, BPW),
        neg3,
        u_embeddings,
        v_embeddings,
    )

    out = pl.pallas_call(
        _finalize_kernel,
        out_shape=jax.ShapeDtypeStruct((1, 1), jnp.float32),
        in_specs=[pl.BlockSpec(memory_space=pltpu.VMEM)],
        out_specs=pl.BlockSpec(memory_space=pltpu.SMEM),
    )(raw.reshape(NW * NCHUNK, BPW))
    return out[0, 0]


# speedup vs baseline: 1.1835x; 1.1835x over previous
import functools

import jax
import jax.numpy as jnp
from jax import lax
from jax.experimental import pallas as pl
from jax.experimental.pallas import tpu as pltpu
from jax.experimental.pallas import tpu_sc as plsc

EMB_DIM = 128
BATCH = 4096
NEG = 20
NCHUNK = NEG + 1
NW = 32
BPW = BATCH // NW
NA = 16
NRING = 4


def _sc_scores(posu2, posv2, neg3, u_table, v_table):
    mesh = plsc.VectorSubcoreMesh(core_axis_name="c", subcore_axis_name="s")

    @functools.partial(
        pl.kernel,
        mesh=mesh,
        out_type=jax.ShapeDtypeStruct((NW, NCHUNK * BPW), jnp.float32),
        compiler_params=pltpu.CompilerParams(needs_layout_passes=False),
        scratch_types=[
            pltpu.VMEM((BPW,), jnp.int32),
            pltpu.VMEM((BPW,), jnp.int32),
            pltpu.VMEM((NEG, BPW), jnp.int32),
            pltpu.VMEM((BPW, EMB_DIM), jnp.float32),
            pltpu.VMEM((BPW, EMB_DIM), jnp.float32),
            pltpu.VMEM((BPW, EMB_DIM), jnp.float32),
            pltpu.VMEM((BPW, EMB_DIM), jnp.float32),
            pltpu.VMEM((BPW, EMB_DIM), jnp.float32),
            pltpu.VMEM((BPW, EMB_DIM), jnp.float32),
            pltpu.VMEM((8, 16, 17), jnp.float32),
            pltpu.VMEM((NCHUNK * BPW,), jnp.float32),
            pltpu.SemaphoreType.DMA,
            pltpu.SemaphoreType.DMA,
            pltpu.SemaphoreType.DMA,
            pltpu.SemaphoreType.DMA,
            pltpu.SemaphoreType.DMA,
            pltpu.SemaphoreType.DMA,
        ],
    )
    def k(posu_hbm, posv_hbm, neg_hbm, u_hbm, v_hbm, out_hbm,
          idxu, idxv, negidx, urows, vrows, nbuf0, nbuf1, nbuf2, nbuf3,
          accs, scores, semu, semv, sem0, sem1, sem2, sem3):
        wid = lax.axis_index("s") * 2 + lax.axis_index("c")

        pltpu.sync_copy(posu_hbm.at[wid], idxu)
        pltpu.sync_copy(posv_hbm.at[wid], idxv)
        pltpu.sync_copy(neg_hbm.at[wid], negidx)

        ucopy = pltpu.make_async_copy(u_hbm.at[idxu], urows, semu)
        vcopy = pltpu.make_async_copy(v_hbm.at[idxv], vrows, semv)
        ucopy.start()
        vcopy.start()

        def ngather(c, buf, sem):
            return pltpu.make_async_copy(v_hbm.at[negidx.at[c]], buf, sem)

        bufs = [nbuf0, nbuf1, nbuf2, nbuf3]
        sems = [sem0, sem1, sem2, sem3]
        for s in range(NRING):
            ngather(s, bufs[s], sems[s]).start()

        with jax.named_scope("uv_wait"):
            ucopy.wait()
            vcopy.wait()

        lane = lax.iota(jnp.int32, 16)

        def col_reduce(slot):
            cols = [
                plsc.load_gather(
                    accs.at[slot], [lane, jnp.full((16,), j, jnp.int32)])
                for j in range(16)
            ]
            while len(cols) > 1:
                cols = [cols[i] + cols[i + 1] for i in range(0, len(cols), 2)]
            return cols[0]

        def tree8(prods):
            s = [prods[2 * i] + prods[2 * i + 1] for i in range(4)]
            return (s[0] + s[1]) + (s[2] + s[3])

        def dot_row(slot, l, uu, buf, r):
            accs[slot, l, pl.ds(0, 16)] = tree8(
                [uu[q] * buf[r, pl.ds(16 * q, 16)] for q in range(8)])

        with jax.named_scope("pos_loop"):
            @plsc.parallel_loop(0, BPW // 16, unroll=2)
            def pos_group(g):
                base = g * 16
                for l in range(16):
                    b = base + l
                    accs[g, l, pl.ds(0, 16)] = tree8([
                        urows[b, pl.ds(16 * q, 16)]
                        * vrows[b, pl.ds(16 * q, 16)]
                        for q in range(8)])
                scores[pl.ds(base, 16)] = col_reduce(g)

        def neg_group_a(c, buf):
            @plsc.parallel_loop(0, BPW // 16, unroll=2)
            def group(g):
                b = 8 * c + g
                uu = [urows[b, pl.ds(16 * q, 16)] for q in range(8)]
                for l in range(16):
                    dot_row(g, l, uu, buf, 16 * g + l)
                scores[pl.ds(BPW + BPW * c + 16 * g, 16)] = col_reduce(g)

        def neg_group_b(c, buf):
            @plsc.parallel_loop(0, BPW // 16, unroll=2)
            def group(g):
                b0 = 4 * (8 * (c - NA) + g)
                for i in range(4):
                    uu = [urows[b0 + i, pl.ds(16 * q, 16)] for q in range(8)]
                    for j in range(4):
                        l = 4 * i + j
                        dot_row(g, l, uu, buf, 16 * g + l)
                scores[pl.ds(BPW + BPW * c + 16 * g, 16)] = col_reduce(g)

        def body_a(i, _):
            c0 = NRING * i
            for s in range(NRING):
                c = c0 + s
                ngather(c, bufs[s], sems[s]).wait()
                neg_group_a(c, bufs[s])

                @pl.when(c + NRING < NEG)
                def _():
                    ngather(c + NRING, bufs[s], sems[s]).start()
            return 0

        with jax.named_scope("neg_a_loop"):
            lax.fori_loop(0, NA // NRING, body_a, 0)

        with jax.named_scope("neg_b"):
            for s in range(NRING):
                c = NA + s
                ngather(c, bufs[s], sems[s]).wait()
                neg_group_b(c, bufs[s])

        with jax.named_scope("writeback"):
            pltpu.sync_copy(scores, out_hbm.at[wid])

    return k(posu2, posv2, neg3, u_table, v_table)


def _finalize_kernel(s_ref, o_ref):
    x = s_ref[...]
    rows = lax.broadcasted_iota(jnp.int32, x.shape, 0)
    is_pos = (rows % NCHUNK) == 0
    xc = jnp.clip(x, -10.0, 10.0)
    p = -jax.nn.log_sigmoid(xc)
    p = -jax.nn.log_sigmoid(jnp.clip(p, -10.0, 10.0))
    n = -jax.nn.log_sigmoid(-xc)
    val = jnp.where(is_pos, p, n)
    o_ref[0, 0] = jnp.sum(val) / BATCH


def kernel(pos_u, pos_v, neg_v, u_embeddings, v_embeddings):
    pos_u = pos_u.astype(jnp.int32)
    pos_v = pos_v.astype(jnp.int32)
    neg_v = neg_v.astype(jnp.int32)

    na = neg_v[:, :16].reshape(NW, NA, BPW)
    nb = neg_v[:, 16:].reshape(NW, NEG - NA, BPW)
    neg3 = jnp.concatenate([na, nb], axis=1)

    raw = _sc_scores(
        pos_u.reshape(NW, BPW),
        pos_v.reshape(NW, BPW),
        neg3,
        u_embeddings,
        v_embeddings,
    )

    out = pl.pallas_call(
        _finalize_kernel,
        out_shape=jax.ShapeDtypeStruct((1, 1), jnp.float32),
        in_specs=[pl.BlockSpec(memory_space=pltpu.VMEM)],
        out_specs=pl.BlockSpec(memory_space=pltpu.SMEM),
    )(raw.reshape(NW * NCHUNK, BPW))
    return out[0, 0]


# speedup vs baseline: 1.1900x; 1.0055x over previous
import functools

import jax
import jax.numpy as jnp
from jax import lax
from jax.experimental import pallas as pl
from jax.experimental.pallas import tpu as pltpu
from jax.experimental.pallas import tpu_sc as plsc

EMB_DIM = 128
BATCH = 4096
NEG = 20
NCHUNK = NEG + 1
NW = 32
BPW = BATCH // NW
NA = 16
NRING = 4


def _sc_scores(posu2, posv2, neg3, u_table, v_table):
    mesh = plsc.VectorSubcoreMesh(core_axis_name="c", subcore_axis_name="s")

    @functools.partial(
        pl.kernel,
        mesh=mesh,
        out_type=jax.ShapeDtypeStruct((NW, 16), jnp.float32),
        compiler_params=pltpu.CompilerParams(needs_layout_passes=False),
        scratch_types=[
            pltpu.VMEM((BPW,), jnp.int32),
            pltpu.VMEM((BPW,), jnp.int32),
            pltpu.VMEM((NEG, BPW), jnp.int32),
            pltpu.VMEM((BPW, EMB_DIM), jnp.float32),
            pltpu.VMEM((BPW, EMB_DIM), jnp.float32),
            pltpu.VMEM((BPW, EMB_DIM), jnp.float32),
            pltpu.VMEM((BPW, EMB_DIM), jnp.float32),
            pltpu.VMEM((BPW, EMB_DIM), jnp.float32),
            pltpu.VMEM((BPW, EMB_DIM), jnp.float32),
            pltpu.VMEM((8, 16, 17), jnp.float32),
            pltpu.VMEM((8, 16), jnp.float32),
            pltpu.VMEM((16,), jnp.float32),
            pltpu.SemaphoreType.DMA,
            pltpu.SemaphoreType.DMA,
            pltpu.SemaphoreType.DMA,
            pltpu.SemaphoreType.DMA,
            pltpu.SemaphoreType.DMA,
            pltpu.SemaphoreType.DMA,
        ],
    )
    def k(posu_hbm, posv_hbm, neg_hbm, u_hbm, v_hbm, out_hbm,
          idxu, idxv, negidx, urows, vrows, nbuf0, nbuf1, nbuf2, nbuf3,
          accs, psums, total16, semu, semv, sem0, sem1, sem2, sem3):
        wid = lax.axis_index("s") * 2 + lax.axis_index("c")

        pltpu.sync_copy(posu_hbm.at[wid], idxu)
        pltpu.sync_copy(posv_hbm.at[wid], idxv)
        pltpu.sync_copy(neg_hbm.at[wid], negidx)

        ucopy = pltpu.make_async_copy(u_hbm.at[idxu], urows, semu)
        vcopy = pltpu.make_async_copy(v_hbm.at[idxv], vrows, semv)
        ucopy.start()
        vcopy.start()

        def ngather(c, buf, sem):
            return pltpu.make_async_copy(v_hbm.at[negidx.at[c]], buf, sem)

        bufs = [nbuf0, nbuf1, nbuf2, nbuf3]
        sems = [sem0, sem1, sem2, sem3]
        for s in range(NRING):
            ngather(s, bufs[s], sems[s]).start()

        with jax.named_scope("uv_wait"):
            ucopy.wait()
            vcopy.wait()

        lane = lax.iota(jnp.int32, 16)

        def col_reduce(slot):
            cols = [
                plsc.load_gather(
                    accs.at[slot], [lane, jnp.full((16,), j, jnp.int32)])
                for j in range(16)
            ]
            while len(cols) > 1:
                cols = [cols[i] + cols[i + 1] for i in range(0, len(cols), 2)]
            return cols[0]

        def tree8(prods):
            s = [prods[2 * i] + prods[2 * i + 1] for i in range(4)]
            return (s[0] + s[1]) + (s[2] + s[3])

        def dot_row(slot, l, uu, buf, r):
            accs[slot, l, pl.ds(0, 16)] = tree8(
                [uu[q] * buf[r, pl.ds(16 * q, 16)] for q in range(8)])

        def ln1p01(y):
            p = jnp.float32(0.0301022476)
            for coef in (-0.1301179303, 0.2833023836, -0.4891557820,
                         0.9990102089, 2.213278e-05):
                p = p * y + jnp.float32(coef)
            return p

        def softplus(x):
            return jnp.maximum(x, 0.0) + ln1p01(jnp.exp(-jnp.abs(x)))

        def pos_loss(res):
            xc = jnp.clip(res, -10.0, 10.0)
            p = softplus(-xc)
            return softplus(-jnp.clip(p, -10.0, 10.0))

        def neg_loss(res):
            return softplus(jnp.clip(res, -10.0, 10.0))

        with jax.named_scope("pos_loop"):
            @plsc.parallel_loop(0, BPW // 16, unroll=2)
            def pos_group(g):
                base = g * 16
                for l in range(16):
                    b = base + l
                    accs[g, l, pl.ds(0, 16)] = tree8([
                        urows[b, pl.ds(16 * q, 16)]
                        * vrows[b, pl.ds(16 * q, 16)]
                        for q in range(8)])
                psums[g] = pos_loss(col_reduce(g))

        def neg_group_a(c, buf):
            @plsc.parallel_loop(0, BPW // 16, unroll=2)
            def group(g):
                b = 8 * c + g
                uu = [urows[b, pl.ds(16 * q, 16)] for q in range(8)]
                for l in range(16):
                    dot_row(g, l, uu, buf, 16 * g + l)
                plsc.addupdate(psums.at[g], neg_loss(col_reduce(g)))

        def neg_group_b(c, buf):
            @plsc.parallel_loop(0, BPW // 16, unroll=2)
            def group(g):
                b0 = 4 * (8 * (c - NA) + g)
                for i in range(4):
                    uu = [urows[b0 + i, pl.ds(16 * q, 16)] for q in range(8)]
                    for j in range(4):
                        l = 4 * i + j
                        dot_row(g, l, uu, buf, 16 * g + l)
                plsc.addupdate(psums.at[g], neg_loss(col_reduce(g)))

        def body_a(i, _):
            c0 = NRING * i
            for s in range(NRING):
                c = c0 + s
                ngather(c, bufs[s], sems[s]).wait()
                neg_group_a(c, bufs[s])

                @pl.when(c + NRING < NEG)
                def _():
                    ngather(c + NRING, bufs[s], sems[s]).start()
            return 0

        with jax.named_scope("neg_a_loop"):
            lax.fori_loop(0, NA // NRING, body_a, 0)

        with jax.named_scope("neg_b"):
            for s in range(NRING):
                c = NA + s
                ngather(c, bufs[s], sems[s]).wait()
                neg_group_b(c, bufs[s])

        with jax.named_scope("writeback"):
            rows = [psums[g] for g in range(8)]
            while len(rows) > 1:
                rows = [rows[i] + rows[i + 1] for i in range(0, len(rows), 2)]
            total16[pl.ds(0, 16)] = rows[0]
            pltpu.sync_copy(total16, out_hbm.at[wid])

    return k(posu2, posv2, neg3, u_table, v_table)


def _finalize_kernel(s_ref, o_ref):
    o_ref[0, 0] = jnp.sum(s_ref[...]) / BATCH


def kernel(pos_u, pos_v, neg_v, u_embeddings, v_embeddings):
    pos_u = pos_u.astype(jnp.int32)
    pos_v = pos_v.astype(jnp.int32)
    neg_v = neg_v.astype(jnp.int32)

    na = neg_v[:, :16].reshape(NW, NA, BPW)
    nb = neg_v[:, 16:].reshape(NW, NEG - NA, BPW)
    neg3 = jnp.concatenate([na, nb], axis=1)

    raw = _sc_scores(
        pos_u.reshape(NW, BPW),
        pos_v.reshape(NW, BPW),
        neg3,
        u_embeddings,
        v_embeddings,
    )

    out = pl.pallas_call(
        _finalize_kernel,
        out_shape=jax.ShapeDtypeStruct((1, 1), jnp.float32),
        in_specs=[pl.BlockSpec(memory_space=pltpu.VMEM)],
        out_specs=pl.BlockSpec(memory_space=pltpu.SMEM),
    )(raw)
    return out[0, 0]
